# baseline (speedup 1.0000x reference)
import jax
import jax.numpy as jnp
from jax import lax
from jax.experimental import pallas as pl
from jax.experimental.pallas import tpu as pltpu
from jax.experimental.pallas import tpu_sc as plsc

ROWS = 128
COLS = 32768
KEEP = 16384
L = 16
ONE_F32_BITS = 0x3F800000

_info = plsc.get_sparse_core_info()
NC = _info.num_cores
NS = _info.num_subcores
NW = NC * NS
RPW = ROWS // NW
UN = 16


def _level_select(hist_v, width, k):
    @plsc.parallel_loop(0, width, L, unroll=UN,
                        carry=(jnp.int32(0), jnp.int32(0), jnp.int32(0)))
    def blk(j, c):
        tot, jb, cbb = c
        s = jnp.sum(hist_v[pl.ds(j, L)])
        tot = tot + s
        below = tot < k
        jb = jnp.where(below, jb + 1, jb)
        cbb = jnp.where(below, tot, cbb)
        return (tot, jb, cbb)

    _, jb, cbb = blk
    h = hist_v[pl.ds(jb * L, L)]
    cum = plsc.cumsum(h) + cbb
    lt = cum < k
    nb = jnp.sum(jnp.where(lt, 1, 0))
    b = jb * L + nb
    cb = jnp.maximum(cbb, jnp.max(jnp.where(lt, cum, 0)))
    return b, cb


def _find_threshold(buf, hist_v, ones, zeros_i):
    @plsc.parallel_loop(0, 2048, L, unroll=UN)
    def _(j):
        hist_v[pl.ds(j, L)] = zeros_i

    @plsc.parallel_loop(0, COLS, L, unroll=UN)
    def _(j):
        a = buf[pl.ds(j, L)] & jnp.int32(0x7FFFFFFF)
        plsc.addupdate_scatter(hist_v, [a >> 20], ones)

    b1, cb1 = _level_select(hist_v, 2048, jnp.int32(KEEP))
    k2 = jnp.int32(KEEP) - cb1

    @plsc.parallel_loop(0, 2048, L, unroll=UN)
    def _(j):
        hist_v[pl.ds(j, L)] = zeros_i

    @plsc.parallel_loop(0, COLS, L, unroll=UN)
    def _(j):
        a = buf[pl.ds(j, L)] & jnp.int32(0x7FFFFFFF)
        m = (a >> 20) == b1
        plsc.addupdate_scatter(
            hist_v, [(a >> 9) & jnp.int32(0x7FF)], ones, mask=m)

    b2, cb2 = _level_select(hist_v, 2048, k2)
    k3 = k2 - cb2
    pref2 = (b1 << 11) | b2

    @plsc.parallel_loop(0, 512, L, unroll=UN)
    def _(j):
        hist_v[pl.ds(j, L)] = zeros_i

    @plsc.parallel_loop(0, COLS, L, unroll=UN)
    def _(j):
        a = buf[pl.ds(j, L)] & jnp.int32(0x7FFFFFFF)
        m = (a >> 9) == pref2
        plsc.addupdate_scatter(hist_v, [a & jnp.int32(0x1FF)], ones, mask=m)

    b3, _ = _level_select(hist_v, 512, k3)
    return (pref2 << 9) | b3


def _tec_body(x_hbm, out_hbm, mask_hbm, buf0, buf1, mrow_v, hist_v,
              in_sem0, in_sem1, out_sem0, out_sem1, mask_sem):
    w = lax.axis_index("s") * NC + lax.axis_index("c")
    ones = jnp.ones((L,), jnp.int32)
    zeros_i = jnp.zeros((L,), jnp.int32)
    bufs = (buf0, buf1)
    in_sems = (in_sem0, in_sem1)
    out_sems = (out_sem0, out_sem1)
    r0 = w * RPW

    pltpu.make_async_copy(x_hbm.at[r0], buf0, in_sem0).start()
    for i in range(RPW):
        buf = bufs[i % 2]
        other = bufs[(i + 1) % 2]
        r = r0 + i
        pltpu.make_async_copy(x_hbm.at[r], buf, in_sems[i % 2]).wait()

        thr = _find_threshold(buf, hist_v, ones, zeros_i)

        if i + 1 < RPW:
            if i >= 1:
                pltpu.make_async_copy(other, out_hbm.at[r - 1],
                                      out_sems[(i + 1) % 2]).wait()
            pltpu.make_async_copy(x_hbm.at[r + 1], other,
                                  in_sems[(i + 1) % 2]).start()
        if i >= 1:
            pltpu.make_async_copy(mrow_v, mask_hbm.at[r - 1], mask_sem).wait()

        @plsc.parallel_loop(0, COLS, L, unroll=UN)
        def _(j):
            v = buf[pl.ds(j, L)]
            m = (v & jnp.int32(0x7FFFFFFF)) >= thr
            buf[pl.ds(j, L)] = jnp.where(m, v, jnp.int32(0))
            mrow_v[pl.ds(j, L)] = jnp.where(m, jnp.int32(ONE_F32_BITS),
                                            jnp.int32(0))

        pltpu.make_async_copy(buf, out_hbm.at[r], out_sems[i % 2]).start()
        pltpu.make_async_copy(mrow_v, mask_hbm.at[r], mask_sem).start()

    pltpu.make_async_copy(bufs[(RPW - 1) % 2], out_hbm.at[r0 + RPW - 1],
                          out_sems[(RPW - 1) % 2]).wait()
    pltpu.make_async_copy(bufs[RPW % 2], out_hbm.at[r0 + RPW - 2],
                          out_sems[RPW % 2]).wait()
    pltpu.make_async_copy(mrow_v, mask_hbm.at[r0 + RPW - 1], mask_sem).wait()


@jax.jit
def kernel(x):
    xb = lax.bitcast_convert_type(x, jnp.int32)
    mesh = plsc.VectorSubcoreMesh(core_axis_name="c", subcore_axis_name="s")
    f = pl.kernel(
        _tec_body,
        out_type=(
            jax.ShapeDtypeStruct((ROWS, COLS), jnp.int32),
            jax.ShapeDtypeStruct((ROWS, COLS), jnp.int32),
        ),
        mesh=mesh,
        scratch_types=[
            pltpu.VMEM((COLS,), jnp.int32),
            pltpu.VMEM((COLS,), jnp.int32),
            pltpu.VMEM((COLS,), jnp.int32),
            pltpu.VMEM((2048,), jnp.int32),
            pltpu.SemaphoreType.DMA,
            pltpu.SemaphoreType.DMA,
            pltpu.SemaphoreType.DMA,
            pltpu.SemaphoreType.DMA,
            pltpu.SemaphoreType.DMA,
        ],
        compiler_params=pltpu.CompilerParams(needs_layout_passes=False),
    )
    out_b, mask_b = f(xb)
    return (lax.bitcast_convert_type(out_b, jnp.float32),
            lax.bitcast_convert_type(mask_b, jnp.float32))


# speedup vs baseline: 1.0318x; 1.0318x over previous
import jax
import jax.numpy as jnp
from jax import lax
from jax.experimental import pallas as pl
from jax.experimental.pallas import tpu as pltpu
from jax.experimental.pallas import tpu_sc as plsc

ROWS = 128
COLS = 32768
KEEP = 16384
L = 16
ONE_F32_BITS = 0x3F800000

_info = plsc.get_sparse_core_info()
NC = _info.num_cores
NS = _info.num_subcores
NW = NC * NS
RPW = ROWS // NW
UN = 4


def _level_select(hist_v, width, k):
    @plsc.parallel_loop(0, width, L, unroll=UN,
                        carry=(jnp.int32(0), jnp.int32(0), jnp.int32(0)))
    def blk(j, c):
        tot, jb, cbb = c
        s = jnp.sum(hist_v[pl.ds(j, L)])
        tot = tot + s
        below = tot < k
        jb = jnp.where(below, jb + 1, jb)
        cbb = jnp.where(below, tot, cbb)
        return (tot, jb, cbb)

    _, jb, cbb = blk
    h = hist_v[pl.ds(jb * L, L)]
    cum = plsc.cumsum(h) + cbb
    lt = cum < k
    nb = jnp.sum(jnp.where(lt, 1, 0))
    b = jb * L + nb
    cb = jnp.maximum(cbb, jnp.max(jnp.where(lt, cum, 0)))
    return b, cb


def _find_threshold(buf, hist_v, ones, zeros_i):
    @plsc.parallel_loop(0, 2048, L, unroll=UN)
    def _(j):
        hist_v[pl.ds(j, L)] = zeros_i

    @plsc.parallel_loop(0, COLS, L, unroll=UN)
    def _(j):
        a = buf[pl.ds(j, L)] & jnp.int32(0x7FFFFFFF)
        plsc.addupdate_scatter(hist_v, [a >> 20], ones)

    b1, cb1 = _level_select(hist_v, 2048, jnp.int32(KEEP))
    k2 = jnp.int32(KEEP) - cb1

    @plsc.parallel_loop(0, 2048, L, unroll=UN)
    def _(j):
        hist_v[pl.ds(j, L)] = zeros_i

    @plsc.parallel_loop(0, COLS, L, unroll=UN)
    def _(j):
        a = buf[pl.ds(j, L)] & jnp.int32(0x7FFFFFFF)
        m = (a >> 20) == b1
        plsc.addupdate_scatter(
            hist_v, [(a >> 9) & jnp.int32(0x7FF)], ones, mask=m)

    b2, cb2 = _level_select(hist_v, 2048, k2)
    k3 = k2 - cb2
    pref2 = (b1 << 11) | b2

    @plsc.parallel_loop(0, 512, L, unroll=UN)
    def _(j):
        hist_v[pl.ds(j, L)] = zeros_i

    @plsc.parallel_loop(0, COLS, L, unroll=UN)
    def _(j):
        a = buf[pl.ds(j, L)] & jnp.int32(0x7FFFFFFF)
        m = (a >> 9) == pref2
        plsc.addupdate_scatter(hist_v, [a & jnp.int32(0x1FF)], ones, mask=m)

    b3, _ = _level_select(hist_v, 512, k3)
    return (pref2 << 9) | b3


def _tec_body(x_hbm, out_hbm, mask_hbm, buf0, buf1, mrow_v, hist_v,
              in_sem0, in_sem1, out_sem0, out_sem1, mask_sem):
    w = lax.axis_index("s") * NC + lax.axis_index("c")
    ones = jnp.ones((L,), jnp.int32)
    zeros_i = jnp.zeros((L,), jnp.int32)
    bufs = (buf0, buf1)
    in_sems = (in_sem0, in_sem1)
    out_sems = (out_sem0, out_sem1)
    r0 = w * RPW

    pltpu.make_async_copy(x_hbm.at[r0], buf0, in_sem0).start()
    for i in range(RPW):
        buf = bufs[i % 2]
        other = bufs[(i + 1) % 2]
        r = r0 + i
        pltpu.make_async_copy(x_hbm.at[r], buf, in_sems[i % 2]).wait()

        thr = _find_threshold(buf, hist_v, ones, zeros_i)

        if i + 1 < RPW:
            if i >= 1:
                pltpu.make_async_copy(other, out_hbm.at[r - 1],
                                      out_sems[(i + 1) % 2]).wait()
            pltpu.make_async_copy(x_hbm.at[r + 1], other,
                                  in_sems[(i + 1) % 2]).start()
        if i >= 1:
            pltpu.make_async_copy(mrow_v, mask_hbm.at[r - 1], mask_sem).wait()

        @plsc.parallel_loop(0, COLS, L, unroll=UN)
        def _(j):
            v = buf[pl.ds(j, L)]
            m = (v & jnp.int32(0x7FFFFFFF)) >= thr
            buf[pl.ds(j, L)] = jnp.where(m, v, jnp.int32(0))
            mrow_v[pl.ds(j, L)] = jnp.where(m, jnp.int32(ONE_F32_BITS),
                                            jnp.int32(0))

        pltpu.make_async_copy(buf, out_hbm.at[r], out_sems[i % 2]).start()
        pltpu.make_async_copy(mrow_v, mask_hbm.at[r], mask_sem).start()

    pltpu.make_async_copy(bufs[(RPW - 1) % 2], out_hbm.at[r0 + RPW - 1],
                          out_sems[(RPW - 1) % 2]).wait()
    pltpu.make_async_copy(bufs[RPW % 2], out_hbm.at[r0 + RPW - 2],
                          out_sems[RPW % 2]).wait()
    pltpu.make_async_copy(mrow_v, mask_hbm.at[r0 + RPW - 1], mask_sem).wait()


@jax.jit
def kernel(x):
    xb = lax.bitcast_convert_type(x, jnp.int32)
    mesh = plsc.VectorSubcoreMesh(core_axis_name="c", subcore_axis_name="s")
    f = pl.kernel(
        _tec_body,
        out_type=(
            jax.ShapeDtypeStruct((ROWS, COLS), jnp.int32),
            jax.ShapeDtypeStruct((ROWS, COLS), jnp.int32),
        ),
        mesh=mesh,
        scratch_types=[
            pltpu.VMEM((COLS,), jnp.int32),
            pltpu.VMEM((COLS,), jnp.int32),
            pltpu.VMEM((COLS,), jnp.int32),
            pltpu.VMEM((2048,), jnp.int32),
            pltpu.SemaphoreType.DMA,
            pltpu.SemaphoreType.DMA,
            pltpu.SemaphoreType.DMA,
            pltpu.SemaphoreType.DMA,
            pltpu.SemaphoreType.DMA,
        ],
        compiler_params=pltpu.CompilerParams(needs_layout_passes=False),
    )
    out_b, mask_b = f(xb)
    return (lax.bitcast_convert_type(out_b, jnp.float32),
            lax.bitcast_convert_type(mask_b, jnp.float32))


# speedup vs baseline: 1.1626x; 1.1267x over previous
import jax
import jax.numpy as jnp
from jax import lax
from jax.experimental import pallas as pl
from jax.experimental.pallas import tpu as pltpu
from jax.experimental.pallas import tpu_sc as plsc

ROWS = 128
COLS = 32768
KEEP = 16384
L = 16
ONE_F32_BITS = 0x3F800000

_info = plsc.get_sparse_core_info()
NC = _info.num_cores
NS = _info.num_subcores
NW = NC * NS
RPW = ROWS // NW
UN = 8


def _level_select(hist_v, width, k):
    @plsc.parallel_loop(0, width, L, unroll=UN,
                        carry=(jnp.int32(0), jnp.int32(0), jnp.int32(0)))
    def blk(j, c):
        tot, jb, cbb = c
        s = jnp.sum(hist_v[pl.ds(j, L)])
        tot = tot + s
        below = tot < k
        jb = jnp.where(below, jb + 1, jb)
        cbb = jnp.where(below, tot, cbb)
        return (tot, jb, cbb)

    _, jb, cbb = blk
    h = hist_v[pl.ds(jb * L, L)]
    cum = plsc.cumsum(h) + cbb
    lt = cum < k
    nb = jnp.sum(jnp.where(lt, 1, 0))
    b = jb * L + nb
    cb = jnp.maximum(cbb, jnp.max(jnp.where(lt, cum, 0)))
    return b, cb


def _find_threshold(buf, hist_v, ones, zeros_i):
    @plsc.parallel_loop(0, 2048, L, unroll=UN)
    def _(j):
        hist_v[pl.ds(j, L)] = zeros_i

    @plsc.parallel_loop(0, COLS, L, unroll=UN)
    def _(j):
        a = buf[pl.ds(j, L)] & jnp.int32(0x7FFFFFFF)
        plsc.addupdate_scatter(hist_v, [a >> 20], ones)

    b1, cb1 = _level_select(hist_v, 2048, jnp.int32(KEEP))
    k2 = jnp.int32(KEEP) - cb1

    @plsc.parallel_loop(0, 2048, L, unroll=UN)
    def _(j):
        hist_v[pl.ds(j, L)] = zeros_i

    @plsc.parallel_loop(0, COLS, L, unroll=UN)
    def _(j):
        a = buf[pl.ds(j, L)] & jnp.int32(0x7FFFFFFF)
        m = (a >> 20) == b1
        plsc.addupdate_scatter(
            hist_v, [(a >> 9) & jnp.int32(0x7FF)], ones, mask=m)

    b2, cb2 = _level_select(hist_v, 2048, k2)
    k3 = k2 - cb2
    pref2 = (b1 << 11) | b2

    @plsc.parallel_loop(0, 512, L, unroll=UN)
    def _(j):
        hist_v[pl.ds(j, L)] = zeros_i

    @plsc.parallel_loop(0, COLS, L, unroll=UN)
    def _(j):
        a = buf[pl.ds(j, L)] & jnp.int32(0x7FFFFFFF)
        m = (a >> 9) == pref2
        plsc.addupdate_scatter(hist_v, [a & jnp.int32(0x1FF)], ones, mask=m)

    b3, _ = _level_select(hist_v, 512, k3)
    return (pref2 << 9) | b3


def _sc_body(x_hbm, thr_hbm, buf0, buf1, thr_v, hist_v, in_sem0, in_sem1):
    w = lax.axis_index("s") * NC + lax.axis_index("c")
    ones = jnp.ones((L,), jnp.int32)
    zeros_i = jnp.zeros((L,), jnp.int32)
    bufs = (buf0, buf1)
    in_sems = (in_sem0, in_sem1)
    r0 = w * RPW

    pltpu.make_async_copy(x_hbm.at[r0], buf0, in_sem0).start()
    pltpu.make_async_copy(x_hbm.at[r0 + 1], buf1, in_sem1).start()
    for i in range(RPW):
        buf = bufs[i % 2]
        r = r0 + i
        pltpu.make_async_copy(x_hbm.at[r], buf, in_sems[i % 2]).wait()

        thr = _find_threshold(buf, hist_v, ones, zeros_i)

        if i + 2 < RPW:
            pltpu.make_async_copy(x_hbm.at[r + 2], buf,
                                  in_sems[i % 2]).start()
        thr_v[pl.ds(0, L)] = jnp.broadcast_to(thr, (L,))
        pltpu.sync_copy(thr_v, thr_hbm.at[r])


def _tc_apply(x_ref, thr_ref, out_ref, mask_ref):
    x = x_ref[...]
    t = thr_ref[...]
    m = jnp.abs(x) >= t
    out_ref[...] = jnp.where(m, x, jnp.float32(0.0))
    mask_ref[...] = m.astype(jnp.float32)


@jax.jit
def kernel(x):
    xb = lax.bitcast_convert_type(x, jnp.int32)
    mesh = plsc.VectorSubcoreMesh(core_axis_name="c", subcore_axis_name="s")
    sc_f = pl.kernel(
        _sc_body,
        out_type=jax.ShapeDtypeStruct((ROWS, L), jnp.int32),
        mesh=mesh,
        scratch_types=[
            pltpu.VMEM((COLS,), jnp.int32),
            pltpu.VMEM((COLS,), jnp.int32),
            pltpu.VMEM((L,), jnp.int32),
            pltpu.VMEM((2048,), jnp.int32),
            pltpu.SemaphoreType.DMA,
            pltpu.SemaphoreType.DMA,
        ],
        compiler_params=pltpu.CompilerParams(needs_layout_passes=False),
    )
    thr_bits = sc_f(xb)
    thr_f = lax.bitcast_convert_type(thr_bits[:, :1], jnp.float32)

    br = 8
    out, mask = pl.pallas_call(
        _tc_apply,
        grid=(ROWS // br,),
        in_specs=[
            pl.BlockSpec((br, COLS), lambda i: (i, 0)),
            pl.BlockSpec((br, 1), lambda i: (i, 0)),
        ],
        out_specs=[
            pl.BlockSpec((br, COLS), lambda i: (i, 0)),
            pl.BlockSpec((br, COLS), lambda i: (i, 0)),
        ],
        out_shape=[
            jax.ShapeDtypeStruct((ROWS, COLS), jnp.float32),
            jax.ShapeDtypeStruct((ROWS, COLS), jnp.float32),
        ],
    )(x, thr_f)
    return (out, mask)


# speedup vs baseline: 1.2834x; 1.1039x over previous
import jax
import jax.numpy as jnp
from jax import lax
from jax.experimental import pallas as pl
from jax.experimental.pallas import tpu as pltpu
from jax.experimental.pallas import tpu_sc as plsc

ROWS = 128
COLS = 32768
KEEP = 16384
L = 16
ONE_F32_BITS = 0x3F800000

_info = plsc.get_sparse_core_info()
NC = _info.num_cores
NS = _info.num_subcores
NW = NC * NS
RPW = ROWS // NW
UN = 8


def _level_select(hist_v, width, k):
    @plsc.parallel_loop(0, width, L, unroll=UN,
                        carry=(jnp.int32(0), jnp.int32(0), jnp.int32(0)))
    def blk(j, c):
        tot, jb, cbb = c
        s = jnp.sum(hist_v[pl.ds(j, L)])
        tot = tot + s
        below = tot < k
        jb = jnp.where(below, jb + 1, jb)
        cbb = jnp.where(below, tot, cbb)
        return (tot, jb, cbb)

    _, jb, cbb = blk
    h = hist_v[pl.ds(jb * L, L)]
    cum = plsc.cumsum(h) + cbb
    lt = cum < k
    nb = jnp.sum(jnp.where(lt, 1, 0))
    b = jb * L + nb
    cb = jnp.maximum(cbb, jnp.max(jnp.where(lt, cum, 0)))
    return b, cb


def _find_threshold(buf, hist_v, ones, zeros_i):
    @plsc.parallel_loop(0, 2048, L, unroll=UN)
    def _(j):
        hist_v[pl.ds(j, L)] = zeros_i

    @plsc.parallel_loop(0, COLS, L, unroll=UN)
    def _(j):
        a = lax.bitcast_convert_type(buf[pl.ds(j, L)],
                                     jnp.int32) & jnp.int32(0x7FFFFFFF)
        plsc.addupdate_scatter(hist_v, [a >> 20], ones)

    b1, cb1 = _level_select(hist_v, 2048, jnp.int32(KEEP))
    k2 = jnp.int32(KEEP) - cb1

    @plsc.parallel_loop(0, 2048, L, unroll=UN)
    def _(j):
        hist_v[pl.ds(j, L)] = zeros_i

    @plsc.parallel_loop(0, COLS, L, unroll=UN)
    def _(j):
        a = lax.bitcast_convert_type(buf[pl.ds(j, L)],
                                     jnp.int32) & jnp.int32(0x7FFFFFFF)
        m = (a >> 20) == b1
        plsc.addupdate_scatter(
            hist_v, [(a >> 9) & jnp.int32(0x7FF)], ones, mask=m)

    b2, cb2 = _level_select(hist_v, 2048, k2)
    k3 = k2 - cb2
    pref2 = (b1 << 11) | b2

    @plsc.parallel_loop(0, 512, L, unroll=UN)
    def _(j):
        hist_v[pl.ds(j, L)] = zeros_i

    @plsc.parallel_loop(0, COLS, L, unroll=UN)
    def _(j):
        a = lax.bitcast_convert_type(buf[pl.ds(j, L)],
                                     jnp.int32) & jnp.int32(0x7FFFFFFF)
        m = (a >> 9) == pref2
        plsc.addupdate_scatter(hist_v, [a & jnp.int32(0x1FF)], ones, mask=m)

    b3, _ = _level_select(hist_v, 512, k3)
    return (pref2 << 9) | b3


def _sc_body(x_hbm, thr_hbm, buf0, buf1, thr_v, hist_v, in_sem0, in_sem1):
    w = lax.axis_index("s") * NC + lax.axis_index("c")
    ones = jnp.ones((L,), jnp.int32)
    zeros_i = jnp.zeros((L,), jnp.int32)
    bufs = (buf0, buf1)
    in_sems = (in_sem0, in_sem1)
    r0 = w * RPW

    pltpu.make_async_copy(x_hbm.at[r0], buf0, in_sem0).start()
    pltpu.make_async_copy(x_hbm.at[r0 + 1], buf1, in_sem1).start()
    for i in range(RPW):
        buf = bufs[i % 2]
        r = r0 + i
        pltpu.make_async_copy(x_hbm.at[r], buf, in_sems[i % 2]).wait()

        thr = _find_threshold(buf, hist_v, ones, zeros_i)

        if i + 2 < RPW:
            pltpu.make_async_copy(x_hbm.at[r + 2], buf,
                                  in_sems[i % 2]).start()
        thr_v[pl.ds(0, L)] = jnp.broadcast_to(thr, (L,))
        pltpu.sync_copy(thr_v, thr_hbm.at[r])


def _tc_apply(x_ref, thr_ref, out_ref, mask_ref):
    x = x_ref[...]
    tb = thr_ref[...]
    t = lax.bitcast_convert_type(tb[:, :1], jnp.float32)
    m = jnp.abs(x) >= t
    out_ref[...] = jnp.where(m, x, jnp.float32(0.0))
    mask_ref[...] = m.astype(jnp.float32)


@jax.jit
def kernel(x):
    mesh = plsc.VectorSubcoreMesh(core_axis_name="c", subcore_axis_name="s")
    sc_f = pl.kernel(
        _sc_body,
        out_type=jax.ShapeDtypeStruct((ROWS, L), jnp.int32),
        mesh=mesh,
        scratch_types=[
            pltpu.VMEM((COLS,), jnp.float32),
            pltpu.VMEM((COLS,), jnp.float32),
            pltpu.VMEM((L,), jnp.int32),
            pltpu.VMEM((2048,), jnp.int32),
            pltpu.SemaphoreType.DMA,
            pltpu.SemaphoreType.DMA,
        ],
        compiler_params=pltpu.CompilerParams(needs_layout_passes=False),
    )
    thr_bits = sc_f(x)

    br = 8
    out, mask = pl.pallas_call(
        _tc_apply,
        grid=(ROWS // br,),
        in_specs=[
            pl.BlockSpec((br, COLS), lambda i: (i, 0)),
            pl.BlockSpec((br, L), lambda i: (i, 0)),
        ],
        out_specs=[
            pl.BlockSpec((br, COLS), lambda i: (i, 0)),
            pl.BlockSpec((br, COLS), lambda i: (i, 0)),
        ],
        out_shape=[
            jax.ShapeDtypeStruct((ROWS, COLS), jnp.float32),
            jax.ShapeDtypeStruct((ROWS, COLS), jnp.float32),
        ],
    )(x, thr_bits)
    return (out, mask)
